# baseline (speedup 1.0000x reference)
import jax
import jax.numpy as jnp
from jax.experimental import pallas as pl

NSAMP = 1024
NEXP = 8192
KT = 32
LSEQ = 200


def _stage_a(ev_ref, t_ref, alpha_ref, mu_ref, bu_ref, exp_ref, prm_ref,
             rate_ref, st_ref, a2_ref):
    tle = prm_ref[0, 0]
    bnd = prm_ref[0, 1]
    beta = prm_ref[0, 2]
    ev = ev_ref[:, :]
    onehot = (ev == jax.lax.broadcasted_iota(jnp.int32, (LSEQ, KT), 1)).astype(jnp.float32)
    a2 = jnp.dot(onehot, alpha_ref[:, :], preferred_element_type=jnp.float32) * 0.1
    a2_ref[:, :] = a2
    tfb = tle + bu_ref[:, :] * (bnd - tle)
    dt = tfb - t_ref[:, :]
    kern = jnp.exp(-beta * jnp.maximum(dt, 0.0)) * (dt > 0.0).astype(jnp.float32)
    excit = jnp.dot(kern, a2, preferred_element_type=jnp.float32)
    lam = jax.nn.softplus(mu_ref[:, :] + excit) + 1e-6
    sums = jnp.sum(lam, axis=1, keepdims=True)
    rid = jax.lax.broadcasted_iota(jnp.int32, (16, 1), 0)
    sums = jnp.where(rid < 10, sums, -jnp.inf)
    rate = jnp.max(sums) * 5.0
    rate_ref[:, :] = jnp.broadcast_to(rate, (1, 1))
    e = -jnp.log1p(-exp_ref[:, :]) / rate
    ii = jax.lax.broadcasted_iota(jnp.int32, (128, 128), 0)
    jj = jax.lax.broadcasted_iota(jnp.int32, (128, 128), 1)
    upper = (ii <= jj).astype(jnp.float32)
    y = jnp.dot(e, upper, preferred_element_type=jnp.float32)
    totals = y[:, 127:128]
    i2 = jax.lax.broadcasted_iota(jnp.int32, (64, 64), 0)
    j2 = jax.lax.broadcasted_iota(jnp.int32, (64, 64), 1)
    lstrict = (j2 < i2).astype(jnp.float32)
    off = jnp.dot(lstrict, totals, preferred_element_type=jnp.float32)
    st_ref[:, :] = y + off + tle


def _stage_b(st_ref, t_ref, a2_ref, mu_ref, prm_ref, thr_ref):
    beta = prm_ref[0, 0]
    rate = prm_ref[0, 1]
    dt = st_ref[:, :] - t_ref[:, :]
    kern = jnp.exp(-beta * jnp.maximum(dt, 0.0)) * (dt > 0.0).astype(jnp.float32)
    excit = jnp.dot(kern, a2_ref[:, :], preferred_element_type=jnp.float32)
    lam = jax.nn.softplus(mu_ref[:, :] + excit) + 1e-6
    thr_ref[:, :] = jnp.sum(lam, axis=1, keepdims=True) / rate


def _stage_c(u_ref, thr_ref, st_ref, prm_ref, out_ref):
    big = prm_ref[0, 1]
    fb = prm_ref[0, 2]
    acc = jnp.where(u_ref[:, :] < thr_ref[:, :], st_ref[:, :], big)
    accmin = jnp.min(acc, axis=1, keepdims=True)
    out_ref[:, :] = jnp.where(accmin < big, accmin, fb)


def kernel(event_seq, time_seq, time_last_event, boundary, bound_u, exp_u,
           unif_numbers, mu, alpha, beta_raw):
    f32 = jnp.float32
    tle = jnp.float32(time_last_event)
    bnd = jnp.float32(boundary)
    beta = jnp.abs(beta_raw[0]) + 0.1

    ev2d = event_seq.reshape(LSEQ, 1).astype(jnp.int32)
    bu16 = jnp.zeros((16, 1), f32).at[:10, 0].set(bound_u)
    exp2d = exp_u.reshape(64, 128)
    mu_row = mu.reshape(1, KT)
    prm_a = jnp.stack([tle, bnd, beta]).reshape(1, 3).astype(f32)

    rate11, st2d, a2 = pl.pallas_call(
        _stage_a,
        out_shape=[
            jax.ShapeDtypeStruct((1, 1), f32),
            jax.ShapeDtypeStruct((64, 128), f32),
            jax.ShapeDtypeStruct((LSEQ, KT), f32),
        ],
    )(ev2d, time_seq, alpha, mu_row, bu16, exp2d, prm_a)

    st_col = st2d.reshape(NEXP, 1)
    prm_b = jnp.stack([beta, rate11[0, 0]]).reshape(1, 2).astype(f32)
    BB = 1024
    thr_col = pl.pallas_call(
        _stage_b,
        grid=(NEXP // BB,),
        in_specs=[
            pl.BlockSpec((BB, 1), lambda i: (i, 0)),
            pl.BlockSpec((1, LSEQ), lambda i: (0, 0)),
            pl.BlockSpec((LSEQ, KT), lambda i: (0, 0)),
            pl.BlockSpec((1, KT), lambda i: (0, 0)),
            pl.BlockSpec((1, 2), lambda i: (0, 0)),
        ],
        out_specs=pl.BlockSpec((BB, 1), lambda i: (i, 0)),
        out_shape=jax.ShapeDtypeStruct((NEXP, 1), f32),
    )(st_col, time_seq, a2, mu_row, prm_b)

    st_row = st2d.reshape(1, NEXP)
    thr_row = thr_col.reshape(1, NEXP)
    rate = rate11[0, 0]
    st_last = st_row[0, NEXP - 1]
    big = st_last + 1.0
    fb = jnp.where(st_last > bnd, st_last, bnd)
    prm_c = jnp.stack([rate, big, fb]).reshape(1, 3).astype(f32)

    RB = 128
    rst_col = pl.pallas_call(
        _stage_c,
        grid=(NSAMP // RB,),
        in_specs=[
            pl.BlockSpec((RB, NEXP), lambda i: (i, 0)),
            pl.BlockSpec((1, NEXP), lambda i: (0, 0)),
            pl.BlockSpec((1, NEXP), lambda i: (0, 0)),
            pl.BlockSpec((1, 3), lambda i: (0, 0)),
        ],
        out_specs=pl.BlockSpec((RB, 1), lambda i: (i, 0)),
        out_shape=jax.ShapeDtypeStruct((NSAMP, 1), f32),
    )(unif_numbers, thr_row, st_row, prm_c)

    rst = rst_col.reshape(NSAMP)
    weights = jnp.full((NSAMP,), 1.0 / NSAMP, f32)
    return rst, weights


# speedup vs baseline: 1.5775x; 1.5775x over previous
import jax
import jax.numpy as jnp
from jax.experimental import pallas as pl

NSAMP = 1024
NEXP = 8192
KT = 32
LSEQ = 200


def _stage_ab(ev_ref, t_ref, alpha_ref, mu_ref, bu_ref, exp_ref, prm_ref,
              rate_ref, st_ref, thr_ref):
    tle = prm_ref[0, 0]
    bnd = prm_ref[0, 1]
    beta = prm_ref[0, 2]
    ev = ev_ref[:, :]
    onehot = (ev == jax.lax.broadcasted_iota(jnp.int32, (LSEQ, KT), 1)).astype(jnp.float32)
    a2 = jnp.dot(onehot, alpha_ref[:, :], preferred_element_type=jnp.float32) * 0.1
    w_row = jnp.exp(-beta * (tle - t_ref[:, :]))
    cvec = jnp.dot(w_row, a2, preferred_element_type=jnp.float32)

    def total_intensity(ebs):
        acc = jnp.zeros_like(ebs)
        for k in range(KT):
            acc = acc + jax.nn.softplus(mu_ref[0, k] + ebs * cvec[0, k])
        return acc + KT * 1e-6

    tfb = tle + bu_ref[:, :] * (bnd - tle)
    sums = total_intensity(jnp.exp(-beta * (tfb - tle)))
    rid = jax.lax.broadcasted_iota(jnp.int32, (16, 1), 0)
    sums = jnp.where(rid < 10, sums, -jnp.inf)
    rate = jnp.max(sums) * 5.0
    rate_ref[:, :] = jnp.broadcast_to(rate, (1, 1))
    e = -jnp.log1p(-exp_ref[:, :]) / rate
    ii = jax.lax.broadcasted_iota(jnp.int32, (128, 128), 0)
    jj = jax.lax.broadcasted_iota(jnp.int32, (128, 128), 1)
    upper = (ii <= jj).astype(jnp.float32)
    y = jnp.dot(e, upper, preferred_element_type=jnp.float32)
    totals = y[:, 127:128]
    i2 = jax.lax.broadcasted_iota(jnp.int32, (64, 64), 0)
    j2 = jax.lax.broadcasted_iota(jnp.int32, (64, 64), 1)
    lstrict = (j2 < i2).astype(jnp.float32)
    off = jnp.dot(lstrict, totals, preferred_element_type=jnp.float32)
    strel = y + off
    st_ref[:, :] = strel + tle
    ti = total_intensity(jnp.exp(-beta * strel))
    thr_ref[:, :] = ti / rate


def _stage_c(u_ref, thr_ref, st_ref, prm_ref, out_ref):
    big = prm_ref[0, 1]
    fb = prm_ref[0, 2]
    acc = jnp.where(u_ref[:, :] < thr_ref[:, :], st_ref[:, :], big)
    accmin = jnp.min(acc, axis=1, keepdims=True)
    out_ref[:, :] = jnp.where(accmin < big, accmin, fb)


def kernel(event_seq, time_seq, time_last_event, boundary, bound_u, exp_u,
           unif_numbers, mu, alpha, beta_raw):
    f32 = jnp.float32
    tle = jnp.float32(time_last_event)
    bnd = jnp.float32(boundary)
    beta = jnp.abs(beta_raw[0]) + 0.1

    ev2d = event_seq.reshape(LSEQ, 1).astype(jnp.int32)
    bu16 = jnp.zeros((16, 1), f32).at[:10, 0].set(bound_u)
    exp2d = exp_u.reshape(64, 128)
    mu_row = mu.reshape(1, KT)
    prm_a = jnp.stack([tle, bnd, beta]).reshape(1, 3).astype(f32)

    rate11, st2d, thr2d = pl.pallas_call(
        _stage_ab,
        out_shape=[
            jax.ShapeDtypeStruct((1, 1), f32),
            jax.ShapeDtypeStruct((64, 128), f32),
            jax.ShapeDtypeStruct((64, 128), f32),
        ],
    )(ev2d, time_seq, alpha, mu_row, bu16, exp2d, prm_a)

    st_row = st2d.reshape(1, NEXP)
    thr_row = thr2d.reshape(1, NEXP)
    rate = rate11[0, 0]
    st_last = st_row[0, NEXP - 1]
    big = st_last + 1.0
    fb = jnp.where(st_last > bnd, st_last, bnd)
    prm_c = jnp.stack([rate, big, fb]).reshape(1, 3).astype(f32)

    RB = 128
    rst_col = pl.pallas_call(
        _stage_c,
        grid=(NSAMP // RB,),
        in_specs=[
            pl.BlockSpec((RB, NEXP), lambda i: (i, 0)),
            pl.BlockSpec((1, NEXP), lambda i: (0, 0)),
            pl.BlockSpec((1, NEXP), lambda i: (0, 0)),
            pl.BlockSpec((1, 3), lambda i: (0, 0)),
        ],
        out_specs=pl.BlockSpec((RB, 1), lambda i: (i, 0)),
        out_shape=jax.ShapeDtypeStruct((NSAMP, 1), f32),
    )(unif_numbers, thr_row, st_row, prm_c)

    rst = rst_col.reshape(NSAMP)
    weights = jnp.full((NSAMP,), 1.0 / NSAMP, f32)
    return rst, weights
